# baseline (speedup 1.0000x reference)
import functools

import jax
import jax.numpy as jnp
from jax import lax
from jax.experimental import pallas as pl
from jax.experimental.pallas import tpu as pltpu
from jax.experimental.pallas import tpu_sc as plsc

NCORES = 2
NTILES = 16
LANES = 16

D = 128
SLAB = 32
NSLAB = D // SLAB
NUNITS = 4 * NSLAB

CHUNK = 128
KROWS = 1
MACRO = CHUNK * KROWS
DEPTH = 5
GAHEAD = 3


def _cdiv(a, b):
    return (a + b - 1) // b


def _splat(vec, e):
    return jnp.take_along_axis(vec, jnp.full((LANES,), e, jnp.int32), axis=0)


def _xt_body(x0r, x1r, x2r, x3r, w_ref, b_ref, out_ref):
    for k, xr in enumerate((x0r, x1r, x2r, x3r)):
        res = jnp.dot(xr[...], w_ref[k], preferred_element_type=jnp.float32)
        out_ref[k] = jnp.maximum(res + b_ref[k][None, :], 0.0)


def _make_xt_call(np_, bm):
    grid = (np_ // bm,)
    return pl.pallas_call(
        _xt_body,
        grid=grid,
        in_specs=[pl.BlockSpec((bm, D), lambda i: (i, 0))] * 4 + [
            pl.BlockSpec((4, D, D), lambda i: (0, 0, 0)),
            pl.BlockSpec((4, D), lambda i: (0, 0)),
        ],
        out_specs=pl.BlockSpec((4, bm, D), lambda i: (0, i, 0)),
        out_shape=jax.ShapeDtypeStruct((4, np_, D), jnp.float32),
        compiler_params=pltpu.CompilerParams(
            dimension_semantics=("arbitrary",),
        ),
    )


def _sc_body(np_, acc_n, ept, table, edata, w_hbm,
             aggr_out, cnt_out, *scratch):
    k = 2
    (acc, cnt_acc) = scratch[0:2]
    ebuf = scratch[k:k + DEPTH]; k += DEPTH
    w_v = scratch[k:k + DEPTH]; k += DEPTH
    rows = scratch[k:k + DEPTH]; k += DEPTH
    scat = scratch[k:k + DEPTH]; k += DEPTH
    (zrow, z128, ones128) = scratch[k:k + 3]; k += 3
    isem = scratch[k:k + DEPTH]; k += DEPTH
    gsem = scratch[k:k + DEPTH]; k += DEPTH
    ssem = scratch[k:k + DEPTH]; k += DEPTH
    csem = scratch[k:k + DEPTH]; k += DEPTH
    zsem = scratch[k]

    c = lax.axis_index("c")
    t = lax.axis_index("s")

    stripe = acc_n // NTILES
    nz = _cdiv(stripe, CHUNK)
    zlast = stripe - CHUNK
    rows_pt = ept // CHUNK
    nmacro = ept // MACRO

    def _init(i, carry):
        for g2 in range(SLAB // LANES):
            zrow[i, pl.ds(g2 * LANES, LANES)] = jnp.zeros((LANES,), jnp.float32)
        return carry
    lax.fori_loop(0, CHUNK, _init, 0)

    def _init1(i, carry):
        z128[pl.ds(i * LANES, LANES)] = jnp.zeros((LANES,), jnp.float32)
        ones128[pl.ds(i * LANES, LANES)] = jnp.ones((LANES,), jnp.float32)
        return carry
    lax.fori_loop(0, CHUNK // LANES, _init1, 0)

    def unit_body(i, carry):
        r = i // 2
        p = 2 * lax.rem(i, 2) + lax.rem(r + c, 2)
        u = 4 * r + p
        base = t * stripe
        off = 4 * r * np_ + p

        def idx_descs(m, s):
            rowb = t * rows_pt + m * KROWS
            ebase = t * ept + m * MACRO
            return (
                pltpu.make_async_copy(
                    edata.at[r, pl.ds(rowb, KROWS)], ebuf[s], isem[s]),
                pltpu.make_async_copy(
                    w_hbm.at[r, pl.ds(ebase, MACRO)], w_v[s], isem[s]),
            )

        def g_descs(s):
            return tuple(
                pltpu.make_async_copy(
                    table.at[ebuf[s].at[j, 0]],
                    rows[s].at[pl.ds(j * CHUNK, CHUNK), :], gsem[s])
                for j in range(KROWS))

        def s_descs(s):
            return tuple(
                pltpu.make_async_copy(
                    rows[s].at[pl.ds(j * CHUNK, CHUNK), :],
                    acc.at[scat[s].at[j]], ssem[s])
                for j in range(KROWS))

        def c_descs(s):
            return tuple(
                pltpu.make_async_copy(
                    ones128, cnt_acc.at[scat[s].at[j]], csem[s])
                for j in range(KROWS))

        def bias_tgt(s):
            def oloop(g, cc2):
                for j in range(KROWS):
                    v = ebuf[s][j, 0, pl.ds(g * LANES, LANES)]
                    ebuf[s][j, 0, pl.ds(g * LANES, LANES)] = v + off
                return cc2
            lax.fori_loop(0, CHUNK // LANES, oloop, 0)

        def zero_desc(j):
            lo = base + jnp.minimum(j * CHUNK, zlast)
            return pltpu.make_async_copy(zrow, acc.at[pl.ds(lo, CHUNK), :], zsem)

        def zero_cnt_desc(j):
            lo = base + jnp.minimum(j * CHUNK, zlast)
            return pltpu.make_async_copy(z128, cnt_acc.at[pl.ds(lo, CHUNK)], zsem)

        lax.fori_loop(0, nz, lambda j, cc: (zero_desc(j).start(), cc)[1], 0)

        @pl.when(p == 0)
        def _():
            lax.fori_loop(0, nz, lambda j, cc: (zero_cnt_desc(j).start(), cc)[1], 0)

        lax.fori_loop(0, nz, lambda j, cc: (zero_desc(j).wait(), cc)[1], 0)

        @pl.when(p == 0)
        def _():
            lax.fori_loop(0, nz, lambda j, cc: (zero_cnt_desc(j).wait(), cc)[1], 0)

        plsc.subcore_barrier()

        for s in range(DEPTH):
            for d in idx_descs(s, s):
                d.start()
        for s in range(GAHEAD):
            for d in idx_descs(s, s):
                d.wait()
            bias_tgt(s)
            for d in g_descs(s):
                d.start()

        LAG = DEPTH - GAHEAD

        def process(m, s):
            so = (s + GAHEAD) % DEPTH

            for d in g_descs(s):
                d.wait()

            @pl.when(m >= LAG)
            def _():
                for d in s_descs(so):
                    d.wait()

                @pl.when(p == 0)
                def _():
                    for d in c_descs(so):
                        d.wait()

            @pl.when(m + GAHEAD < nmacro)
            def _():
                for d in idx_descs(m + GAHEAD, so):
                    d.wait()
                bias_tgt(so)
                for d in g_descs(so):
                    d.start()

            for j in range(KROWS):
                for g2 in range(CHUNK // LANES):
                    scat[s][j, pl.ds(g2 * LANES, LANES)] = \
                        ebuf[s][j, 1, pl.ds(g2 * LANES, LANES)]

            @pl.when(m + DEPTH < nmacro)
            def _():
                for d in idx_descs(m + DEPTH, s):
                    d.start()

            def gloop(g, cc2):
                w_vec = w_v[s][pl.ds(g * LANES, LANES)]
                for e in range(LANES):
                    sp_ = _splat(w_vec, e)
                    q = g * LANES + e
                    a0 = rows[s][q, pl.ds(0, LANES)]
                    a1 = rows[s][q, pl.ds(LANES, LANES)]
                    rows[s][q, pl.ds(0, LANES)] = a0 * sp_
                    rows[s][q, pl.ds(LANES, LANES)] = a1 * sp_
                return cc2
            lax.fori_loop(0, MACRO // LANES, gloop, 0)

            for d in s_descs(s):
                d.start(add=True)

            @pl.when(p == 0)
            def _():
                for d in c_descs(s):
                    d.start(add=True)

        def block(q, cc):
            m0 = DEPTH * q
            for k2 in range(DEPTH):
                process(m0 + k2, k2)
            return cc
        lax.fori_loop(0, nmacro // DEPTH, block, 0)

        for mm in range(nmacro - LAG, nmacro):
            for d in s_descs(mm % DEPTH):
                d.wait()

        @pl.when(p == 0)
        def _():
            for mm in range(nmacro - LAG, nmacro):
                for d in c_descs(mm % DEPTH):
                    d.wait()

        plsc.subcore_barrier()

        def wout_desc(j):
            lo = base + jnp.minimum(j * CHUNK, zlast)
            return pltpu.make_async_copy(
                acc.at[pl.ds(lo, CHUNK), :],
                aggr_out.at[r, pl.ds(lo, CHUNK), pl.ds(p * SLAB, SLAB)], zsem)

        def wout_cnt_desc(j):
            lo = base + jnp.minimum(j * CHUNK, zlast)
            return pltpu.make_async_copy(
                cnt_acc.at[pl.ds(lo, CHUNK)],
                cnt_out.at[r, pl.ds(lo, CHUNK)], zsem)

        lax.fori_loop(0, nz, lambda j, cc: (wout_desc(j).start(), cc)[1], 0)

        @pl.when(p == 0)
        def _():
            lax.fori_loop(0, nz, lambda j, cc: (wout_cnt_desc(j).start(), cc)[1], 0)

        lax.fori_loop(0, nz, lambda j, cc: (wout_desc(j).wait(), cc)[1], 0)

        @pl.when(p == 0)
        def _():
            lax.fori_loop(0, nz, lambda j, cc: (wout_cnt_desc(j).wait(), cc)[1], 0)
        return carry

    lax.fori_loop(0, NUNITS // NCORES, unit_body, 0)


def _make_sc_call(np_, acc_n, ept):
    mesh = plsc.VectorSubcoreMesh(
        core_axis_name="c", subcore_axis_name="s",
        num_cores=NCORES, num_subcores=NTILES)
    return pl.kernel(
        functools.partial(_sc_body, np_, acc_n, ept),
        out_type=[
            jax.ShapeDtypeStruct((4, np_, D), jnp.float32),
            jax.ShapeDtypeStruct((4, np_), jnp.float32),
        ],
        mesh=mesh,
        compiler_params=pltpu.CompilerParams(use_tc_tiling_on_sc=False),
        scratch_types=(
            [
                pltpu.VMEM_SHARED((acc_n, SLAB), jnp.float32),
                pltpu.VMEM_SHARED((acc_n,), jnp.float32),
            ]
            + [pltpu.VMEM((KROWS, 2, CHUNK), jnp.int32) for _ in range(DEPTH)]
            + [pltpu.VMEM((MACRO,), jnp.float32) for _ in range(DEPTH)]
            + [pltpu.VMEM((MACRO, SLAB), jnp.float32) for _ in range(DEPTH)]
            + [pltpu.VMEM((KROWS, CHUNK), jnp.int32) for _ in range(DEPTH)]
            + [
                pltpu.VMEM((CHUNK, SLAB), jnp.float32),
                pltpu.VMEM((CHUNK,), jnp.float32),
                pltpu.VMEM((CHUNK,), jnp.float32),
            ]
            + [pltpu.SemaphoreType.DMA for _ in range(DEPTH)]
            + [pltpu.SemaphoreType.DMA for _ in range(DEPTH)]
            + [pltpu.SemaphoreType.DMA for _ in range(DEPTH)]
            + [pltpu.SemaphoreType.DMA for _ in range(DEPTH)]
            + [pltpu.SemaphoreType.DMA]
        ),
    )


def _final_body(ag_ref, cnt_ref, xn_ref, u1_ref, u2_ref,
                wo1_ref, wo2_ref, bo_ref, out_ref):
    xn = xn_ref[...]
    zn = jnp.dot(xn, u2_ref[...], preferred_element_type=jnp.float32)
    aggs = []
    scores = []
    for r in range(4):
        cnt = jnp.maximum(cnt_ref[r], 1.0)
        a = ag_ref[r] / cnt[:, None]
        z = jnp.dot(a, u1_ref[...], preferred_element_type=jnp.float32) + zn
        z = jnp.where(z >= 0.0, z, 0.01 * z)
        aggs.append(a)
        scores.append(jnp.exp(z))
    ssum = scores[0] + scores[1] + scores[2] + scores[3]
    comb = aggs[0] * (scores[0] / ssum)
    for r in range(1, 4):
        comb = comb + aggs[r] * (scores[r] / ssum)
    h = (jnp.dot(xn, wo1_ref[...], preferred_element_type=jnp.float32)
         + jnp.dot(comb, wo2_ref[...], preferred_element_type=jnp.float32)
         + bo_ref[...])
    h = jnp.maximum(h, 0.0)
    nrm = jnp.sqrt(jnp.sum(h * h, axis=1, keepdims=True))
    out_ref[...] = h / jnp.maximum(nrm, 1e-12)


def _make_final_call(n, np_, bn):
    grid = (_cdiv(n, bn),)
    return pl.pallas_call(
        _final_body,
        grid=grid,
        in_specs=[
            pl.BlockSpec((4, bn, D), lambda i: (0, i, 0)),
            pl.BlockSpec((4, bn), lambda i: (0, i)),
            pl.BlockSpec((bn, D), lambda i: (i, 0)),
            pl.BlockSpec((D, 1), lambda i: (0, 0)),
            pl.BlockSpec((D, 1), lambda i: (0, 0)),
            pl.BlockSpec((D, D), lambda i: (0, 0)),
            pl.BlockSpec((D, D), lambda i: (0, 0)),
            pl.BlockSpec((1, D), lambda i: (0, 0)),
        ],
        out_specs=pl.BlockSpec((bn, D), lambda i: (i, 0)),
        out_shape=jax.ShapeDtypeStruct((n, D), jnp.float32),
        compiler_params=pltpu.CompilerParams(
            dimension_semantics=("arbitrary",),
        ),
    )


def kernel(x0, x1, x2, x3, e0, e1, e2, e3, w0, w1, w2, w3, x_node, num_node,
           W_a, b_a, W_p, b_p, W_t, b_t, W_c, b_c, u, W_out, b_out):
    n, d = x_node.shape
    assert d == D
    e = e0.shape[1]

    np_ = _cdiv(n, NTILES * 128) * NTILES * 128
    acc_n = _cdiv(n, NTILES * 8) * NTILES * 8
    ept = _cdiv(e, NTILES * DEPTH * MACRO) * DEPTH * MACRO
    epad = NTILES * ept

    ws_mat = jnp.stack([W_a, W_p, W_t, W_c])
    bs = jnp.stack([b_a, b_p, b_t, b_c])

    pad = epad - e
    ar = jnp.arange(pad, dtype=jnp.int32)
    pad_src = n + ar % (acc_n - n)
    pad_tgt = ar % 64
    recs = []
    for er in (e0, e1, e2, e3):
        src = jnp.concatenate([er[0], pad_src])
        tgt4 = jnp.concatenate([er[1], pad_tgt]) * 4
        recs.append(jnp.stack([tgt4.reshape(-1, CHUNK),
                               src.reshape(-1, CHUNK)], axis=1))
    edata = jnp.stack(recs)
    w_hbm = jnp.stack([
        jnp.concatenate([wr, jnp.zeros((pad,), jnp.float32)])
        for wr in (w0, w1, w2, w3)
    ])

    tables = _make_xt_call(np_, np_ // 16)(x0, x1, x2, x3, ws_mat, bs)
    table_flat = tables.reshape(4 * NSLAB * np_, SLAB)

    aggr4, cnts = _make_sc_call(np_, acc_n, ept)(
        table_flat, edata, w_hbm)

    u1 = u[:D]
    u2 = u[D:]
    wo1 = W_out[:D]
    wo2 = W_out[D:]
    return _make_final_call(n, np_, np_ // 16)(
        aggr4, cnts, x_node, u1, u2, wo1, wo2, b_out.reshape(1, D))


# speedup vs baseline: 1.0340x; 1.0340x over previous
import functools

import jax
import jax.numpy as jnp
from jax import lax
from jax.experimental import pallas as pl
from jax.experimental.pallas import tpu as pltpu
from jax.experimental.pallas import tpu_sc as plsc

NCORES = 2
NTILES = 16
LANES = 16

D = 128
SLAB = 32
NSLAB = D // SLAB
NUNITS = 4 * NSLAB

CHUNK = 128
KROWS = 1
MACRO = CHUNK * KROWS
DEPTH = 5
GAHEAD = 3


def _cdiv(a, b):
    return (a + b - 1) // b


def _splat(vec, e):
    return jnp.take_along_axis(vec, jnp.full((LANES,), e, jnp.int32), axis=0)


def _xt_body(x0r, x1r, x2r, x3r, w_ref, b_ref, out_ref):
    for k, xr in enumerate((x0r, x1r, x2r, x3r)):
        res = jnp.dot(xr[...], w_ref[k], preferred_element_type=jnp.float32)
        out_ref[k] = jnp.maximum(res + b_ref[k][None, :], 0.0)


def _make_xt_call(np_, bm):
    grid = (np_ // bm,)
    return pl.pallas_call(
        _xt_body,
        grid=grid,
        in_specs=[pl.BlockSpec((bm, D), lambda i: (i, 0))] * 4 + [
            pl.BlockSpec((4, D, D), lambda i: (0, 0, 0)),
            pl.BlockSpec((4, D), lambda i: (0, 0)),
        ],
        out_specs=pl.BlockSpec((4, bm, D), lambda i: (0, i, 0)),
        out_shape=jax.ShapeDtypeStruct((4, np_, D), jnp.float32),
        compiler_params=pltpu.CompilerParams(
            dimension_semantics=("arbitrary",),
        ),
    )


def _sc_body(np_, acc_n, ept, table, tgt_hbm, src_hbm, w_hbm,
             aggr_out, cnt_out, *scratch):
    k = 2
    (acc, cnt_acc) = scratch[0:2]
    tgt = scratch[k:k + DEPTH]; k += DEPTH
    src = scratch[k:k + DEPTH]; k += DEPTH
    w_v = scratch[k:k + DEPTH]; k += DEPTH
    rows = scratch[k:k + DEPTH]; k += DEPTH
    scat = scratch[k:k + DEPTH]; k += DEPTH
    (zrow, z128, ones128) = scratch[k:k + 3]; k += 3
    isem = scratch[k:k + DEPTH]; k += DEPTH
    gsem = scratch[k:k + DEPTH]; k += DEPTH
    ssem = scratch[k:k + DEPTH]; k += DEPTH
    csem = scratch[k:k + DEPTH]; k += DEPTH
    zsem = scratch[k]

    c = lax.axis_index("c")
    t = lax.axis_index("s")

    stripe = acc_n // NTILES
    nz = _cdiv(stripe, CHUNK)
    zlast = stripe - CHUNK
    rows_pt = ept // CHUNK
    nmacro = ept // MACRO

    def _init(i, carry):
        for g2 in range(SLAB // LANES):
            zrow[i, pl.ds(g2 * LANES, LANES)] = jnp.zeros((LANES,), jnp.float32)
        return carry
    lax.fori_loop(0, CHUNK, _init, 0)

    def _init1(i, carry):
        z128[pl.ds(i * LANES, LANES)] = jnp.zeros((LANES,), jnp.float32)
        ones128[pl.ds(i * LANES, LANES)] = jnp.ones((LANES,), jnp.float32)
        return carry
    lax.fori_loop(0, CHUNK // LANES, _init1, 0)

    def unit_body(i, carry):
        r = i // 2
        p = 2 * lax.rem(i, 2) + lax.rem(r + c, 2)
        u = 4 * r + p
        base = t * stripe
        off = 4 * r * np_ + p

        def idx_descs(m, s):
            rowb = t * rows_pt + m * KROWS
            ebase = t * ept + m * MACRO
            return (
                pltpu.make_async_copy(
                    tgt_hbm.at[r, pl.ds(ebase, MACRO)], tgt[s], isem[s]),
                pltpu.make_async_copy(
                    w_hbm.at[r, pl.ds(ebase, MACRO)], w_v[s], isem[s]),
                pltpu.make_async_copy(
                    src_hbm.at[r, 0, pl.ds(rowb, KROWS), :], src[s], isem[s]),
            )

        def g_descs(s):
            return tuple(
                pltpu.make_async_copy(
                    table.at[tgt[s].at[pl.ds(j * CHUNK, CHUNK)]],
                    rows[s].at[pl.ds(j * CHUNK, CHUNK), :], gsem[s])
                for j in range(KROWS))

        def s_descs(s):
            return tuple(
                pltpu.make_async_copy(
                    rows[s].at[pl.ds(j * CHUNK, CHUNK), :],
                    acc.at[scat[s].at[j]], ssem[s])
                for j in range(KROWS))

        def c_descs(s):
            return tuple(
                pltpu.make_async_copy(
                    ones128, cnt_acc.at[scat[s].at[j]], csem[s])
                for j in range(KROWS))

        def bias_tgt(s):
            def oloop(g, cc2):
                v = tgt[s][pl.ds(g * LANES, LANES)]
                tgt[s][pl.ds(g * LANES, LANES)] = v * 4 + off
                return cc2
            lax.fori_loop(0, MACRO // LANES, oloop, 0)

        def zero_desc(j):
            lo = base + jnp.minimum(j * CHUNK, zlast)
            return pltpu.make_async_copy(zrow, acc.at[pl.ds(lo, CHUNK), :], zsem)

        def zero_cnt_desc(j):
            lo = base + jnp.minimum(j * CHUNK, zlast)
            return pltpu.make_async_copy(z128, cnt_acc.at[pl.ds(lo, CHUNK)], zsem)

        lax.fori_loop(0, nz, lambda j, cc: (zero_desc(j).start(), cc)[1], 0)

        @pl.when(p == 0)
        def _():
            lax.fori_loop(0, nz, lambda j, cc: (zero_cnt_desc(j).start(), cc)[1], 0)

        lax.fori_loop(0, nz, lambda j, cc: (zero_desc(j).wait(), cc)[1], 0)

        @pl.when(p == 0)
        def _():
            lax.fori_loop(0, nz, lambda j, cc: (zero_cnt_desc(j).wait(), cc)[1], 0)

        plsc.subcore_barrier()

        for s in range(DEPTH):
            for d in idx_descs(s, s):
                d.start()
        for s in range(GAHEAD):
            for d in idx_descs(s, s):
                d.wait()
            bias_tgt(s)
            for d in g_descs(s):
                d.start()

        LAG = DEPTH - GAHEAD

        def process(m, s):
            so = (s + GAHEAD) % DEPTH

            for d in g_descs(s):
                d.wait()

            @pl.when(m >= LAG)
            def _():
                for d in s_descs(so):
                    d.wait()

                @pl.when(p == 0)
                def _():
                    for d in c_descs(so):
                        d.wait()

            @pl.when(m + GAHEAD < nmacro)
            def _():
                for d in idx_descs(m + GAHEAD, so):
                    d.wait()
                bias_tgt(so)
                for d in g_descs(so):
                    d.start()

            for j in range(KROWS):
                for g2 in range(CHUNK // LANES):
                    scat[s][j, pl.ds(g2 * LANES, LANES)] = \
                        src[s][j, pl.ds(g2 * LANES, LANES)]

            @pl.when(m + DEPTH < nmacro)
            def _():
                for d in idx_descs(m + DEPTH, s):
                    d.start()

            def gloop(g, cc2):
                w_vec = w_v[s][pl.ds(g * LANES, LANES)]
                for e in range(LANES):
                    sp_ = _splat(w_vec, e)
                    q = g * LANES + e
                    a0 = rows[s][q, pl.ds(0, LANES)]
                    a1 = rows[s][q, pl.ds(LANES, LANES)]
                    rows[s][q, pl.ds(0, LANES)] = a0 * sp_
                    rows[s][q, pl.ds(LANES, LANES)] = a1 * sp_
                return cc2
            lax.fori_loop(0, MACRO // LANES, gloop, 0)

            for d in s_descs(s):
                d.start(add=True)

            @pl.when(p == 0)
            def _():
                for d in c_descs(s):
                    d.start(add=True)

        def block(q, cc):
            m0 = DEPTH * q
            for k2 in range(DEPTH):
                process(m0 + k2, k2)
            return cc
        lax.fori_loop(0, nmacro // DEPTH, block, 0)

        for mm in range(nmacro - LAG, nmacro):
            for d in s_descs(mm % DEPTH):
                d.wait()

        @pl.when(p == 0)
        def _():
            for mm in range(nmacro - LAG, nmacro):
                for d in c_descs(mm % DEPTH):
                    d.wait()

        plsc.subcore_barrier()

        def wout_desc(j):
            lo = base + jnp.minimum(j * CHUNK, zlast)
            return pltpu.make_async_copy(
                acc.at[pl.ds(lo, CHUNK), :],
                aggr_out.at[r, pl.ds(lo, CHUNK), pl.ds(p * SLAB, SLAB)], zsem)

        def wout_cnt_desc(j):
            lo = base + jnp.minimum(j * CHUNK, zlast)
            return pltpu.make_async_copy(
                cnt_acc.at[pl.ds(lo, CHUNK)],
                cnt_out.at[r, pl.ds(lo, CHUNK)], zsem)

        lax.fori_loop(0, nz, lambda j, cc: (wout_desc(j).start(), cc)[1], 0)

        @pl.when(p == 0)
        def _():
            lax.fori_loop(0, nz, lambda j, cc: (wout_cnt_desc(j).start(), cc)[1], 0)

        lax.fori_loop(0, nz, lambda j, cc: (wout_desc(j).wait(), cc)[1], 0)

        @pl.when(p == 0)
        def _():
            lax.fori_loop(0, nz, lambda j, cc: (wout_cnt_desc(j).wait(), cc)[1], 0)
        return carry

    lax.fori_loop(0, NUNITS // NCORES, unit_body, 0)


def _make_sc_call(np_, acc_n, ept):
    mesh = plsc.VectorSubcoreMesh(
        core_axis_name="c", subcore_axis_name="s",
        num_cores=NCORES, num_subcores=NTILES)
    return pl.kernel(
        functools.partial(_sc_body, np_, acc_n, ept),
        out_type=[
            jax.ShapeDtypeStruct((4, np_, D), jnp.float32),
            jax.ShapeDtypeStruct((4, np_), jnp.float32),
        ],
        mesh=mesh,
        compiler_params=pltpu.CompilerParams(use_tc_tiling_on_sc=False),
        scratch_types=(
            [
                pltpu.VMEM_SHARED((acc_n, SLAB), jnp.float32),
                pltpu.VMEM_SHARED((acc_n,), jnp.float32),
            ]
            + [pltpu.VMEM((MACRO,), jnp.int32) for _ in range(DEPTH)]
            + [pltpu.VMEM((KROWS, CHUNK), jnp.int32) for _ in range(DEPTH)]
            + [pltpu.VMEM((MACRO,), jnp.float32) for _ in range(DEPTH)]
            + [pltpu.VMEM((MACRO, SLAB), jnp.float32) for _ in range(DEPTH)]
            + [pltpu.VMEM((KROWS, CHUNK), jnp.int32) for _ in range(DEPTH)]
            + [
                pltpu.VMEM((CHUNK, SLAB), jnp.float32),
                pltpu.VMEM((CHUNK,), jnp.float32),
                pltpu.VMEM((CHUNK,), jnp.float32),
            ]
            + [pltpu.SemaphoreType.DMA for _ in range(DEPTH)]
            + [pltpu.SemaphoreType.DMA for _ in range(DEPTH)]
            + [pltpu.SemaphoreType.DMA for _ in range(DEPTH)]
            + [pltpu.SemaphoreType.DMA for _ in range(DEPTH)]
            + [pltpu.SemaphoreType.DMA]
        ),
    )


def _final_body(ag_ref, cnt_ref, xn_ref, u1_ref, u2_ref,
                wo1_ref, wo2_ref, bo_ref, out_ref):
    xn = xn_ref[...]
    zn = jnp.dot(xn, u2_ref[...], preferred_element_type=jnp.float32)
    aggs = []
    scores = []
    for r in range(4):
        cnt = jnp.maximum(cnt_ref[r], 1.0)
        a = ag_ref[r] / cnt[:, None]
        z = jnp.dot(a, u1_ref[...], preferred_element_type=jnp.float32) + zn
        z = jnp.where(z >= 0.0, z, 0.01 * z)
        aggs.append(a)
        scores.append(jnp.exp(z))
    ssum = scores[0] + scores[1] + scores[2] + scores[3]
    comb = aggs[0] * (scores[0] / ssum)
    for r in range(1, 4):
        comb = comb + aggs[r] * (scores[r] / ssum)
    h = (jnp.dot(xn, wo1_ref[...], preferred_element_type=jnp.float32)
         + jnp.dot(comb, wo2_ref[...], preferred_element_type=jnp.float32)
         + bo_ref[...])
    h = jnp.maximum(h, 0.0)
    nrm = jnp.sqrt(jnp.sum(h * h, axis=1, keepdims=True))
    out_ref[...] = h / jnp.maximum(nrm, 1e-12)


def _make_final_call(n, np_, bn):
    grid = (_cdiv(n, bn),)
    return pl.pallas_call(
        _final_body,
        grid=grid,
        in_specs=[
            pl.BlockSpec((4, bn, D), lambda i: (0, i, 0)),
            pl.BlockSpec((4, bn), lambda i: (0, i)),
            pl.BlockSpec((bn, D), lambda i: (i, 0)),
            pl.BlockSpec((D, 1), lambda i: (0, 0)),
            pl.BlockSpec((D, 1), lambda i: (0, 0)),
            pl.BlockSpec((D, D), lambda i: (0, 0)),
            pl.BlockSpec((D, D), lambda i: (0, 0)),
            pl.BlockSpec((1, D), lambda i: (0, 0)),
        ],
        out_specs=pl.BlockSpec((bn, D), lambda i: (i, 0)),
        out_shape=jax.ShapeDtypeStruct((n, D), jnp.float32),
        compiler_params=pltpu.CompilerParams(
            dimension_semantics=("arbitrary",),
        ),
    )


def kernel(x0, x1, x2, x3, e0, e1, e2, e3, w0, w1, w2, w3, x_node, num_node,
           W_a, b_a, W_p, b_p, W_t, b_t, W_c, b_c, u, W_out, b_out):
    n, d = x_node.shape
    assert d == D
    e = e0.shape[1]

    np_ = _cdiv(n, NTILES * 128) * NTILES * 128
    acc_n = _cdiv(n, NTILES * 8) * NTILES * 8
    ept = _cdiv(e, NTILES * DEPTH * MACRO) * DEPTH * MACRO
    epad = NTILES * ept

    ws_mat = jnp.stack([W_a, W_p, W_t, W_c])
    bs = jnp.stack([b_a, b_p, b_t, b_c])

    pad = epad - e
    ar = jnp.arange(pad, dtype=jnp.int32)
    pad_src = n + ar % (acc_n - n)
    pad_tgt = ar % 64
    srcs, tgts = [], []
    for er in (e0, e1, e2, e3):
        srcs.append(jnp.concatenate([er[0], pad_src]))
        tgts.append(jnp.concatenate([er[1], pad_tgt]))
    src_hbm = jnp.stack(srcs).reshape(4, 1, epad // CHUNK, CHUNK)
    tgt_hbm = jnp.stack(tgts)
    w_hbm = jnp.stack([
        jnp.concatenate([wr, jnp.zeros((pad,), jnp.float32)])
        for wr in (w0, w1, w2, w3)
    ])

    tables = _make_xt_call(np_, np_ // 16)(x0, x1, x2, x3, ws_mat, bs)
    table_flat = tables.reshape(4 * NSLAB * np_, SLAB)

    aggr4, cnts = _make_sc_call(np_, acc_n, ept)(
        table_flat, tgt_hbm, src_hbm, w_hbm)

    u1 = u[:D]
    u2 = u[D:]
    wo1 = W_out[:D]
    wo2 = W_out[D:]
    return _make_final_call(n, np_, np_ // 16)(
        aggr4, cnts, x_node, u1, u2, wo1, wo2, b_out.reshape(1, D))
